# Initial kernel scaffold; baseline (speedup 1.0000x reference)
import jax
import jax.numpy as jnp
from jax.experimental import pallas as pl


def kernel(time, index1, index2, residuals, means, stds, emb1, emb2, W1, b1, Wm, bm, Ws, bs):
    raise NotImplementedError("write your pallas kernel here")


# speedup vs baseline: 8.5446x; 8.5446x over previous
import functools

import jax
import jax.numpy as jnp
from jax import lax
from jax.experimental import pallas as pl
from jax.experimental.pallas import tpu as pltpu
from jax.experimental.pallas import tpu_sc as plsc

TT = 2
S1 = 2048
S2 = 2048
BQ = 4096
EMB = 256
HID = 64
KN = 20
RB = 256
NW = 32
QPW = BQ // NW
GPW = QPW * KN
NCH = GPW // 128
ROWDIV = S2 // 16
F32 = jnp.float32
I32 = jnp.int32


def _sc_qgather_body(i1_hbm, i2_hbm, emb1_hbm, emb2_hbm, q1_hbm, q2_hbm,
                     idx_v, rows_v, sem):
    wid = lax.axis_index("s") * 2 + lax.axis_index("c")
    base = wid * QPW
    pltpu.sync_copy(i1_hbm.at[pl.ds(base, QPW)], idx_v)
    pltpu.async_copy(emb1_hbm.at[idx_v], rows_v, sem).wait()
    pltpu.sync_copy(rows_v, q1_hbm.at[pl.ds(base, QPW)])
    pltpu.sync_copy(i2_hbm.at[pl.ds(base, QPW)], idx_v)
    pltpu.async_copy(emb2_hbm.at[idx_v], rows_v, sem).wait()
    pltpu.sync_copy(rows_v, q2_hbm.at[pl.ds(base, QPW)])


def _sc_qgather(index1, index2, emb1, emb2):
    mesh = plsc.VectorSubcoreMesh(core_axis_name="c", subcore_axis_name="s")
    fn = pl.kernel(
        _sc_qgather_body,
        mesh=mesh,
        out_type=(jax.ShapeDtypeStruct((BQ, EMB), F32),
                  jax.ShapeDtypeStruct((BQ, EMB), F32)),
        scratch_types=[pltpu.VMEM((QPW,), I32),
                       pltpu.VMEM((QPW, EMB), F32),
                       pltpu.SemaphoreType.DMA],
    )
    return fn(index1, index2, emb1, emb2)


def _topk_side(q, e, self_idx):
    qn = jnp.sum(q * q, axis=1, keepdims=True)
    en = jnp.sum(e * e, axis=1)[None, :]
    g = lax.dot_general(q, e, (((1,), (1,)), ((), ())),
                        preferred_element_type=F32)
    d2 = qn + en - 2.0 * g
    sim = jnp.sqrt(jnp.maximum(d2, 0.0)) + 0.001
    w = jnp.exp(-sim)
    cols = lax.broadcasted_iota(I32, w.shape, 1)
    w = jnp.where(cols == self_idx[:, None], -1.0, w)
    ws, idxs = [], []
    for _ in range(KN):
        m = jnp.max(w, axis=1)
        a = jnp.min(jnp.where(w == m[:, None], cols, w.shape[1]), axis=1)
        ws.append(m)
        idxs.append(a)
        w = jnp.where(cols == a[:, None], -1.0, w)
    return jnp.stack(ws, axis=1), jnp.stack(idxs, axis=1)


def _tc_topk_body(time_ref, i1_ref, i2_ref, q1_ref, q2_ref, e1_ref, e2_ref,
                  w1_ref, l1_ref, w2_ref, l2_ref, lm_ref):
    t = time_ref[...]
    i1 = i1_ref[...]
    i2 = i2_ref[...]
    w1, n1 = _topk_side(q1_ref[...], e1_ref[...], i1)
    w2, n2 = _topk_side(q2_ref[...], e2_ref[...], i2)
    tb = (t * (S1 * S2))[:, None]
    w1_ref[...] = w1
    l1_ref[...] = tb + n1 * S2 + i2[:, None]
    w2_ref[...] = w2
    l2_ref[...] = tb + (i1 * S2)[:, None] + n2
    lm_ref[...] = t * (S1 * S2) + i1 * S2 + i2


def _tc_topk(time, index1, index2, q1, q2, emb1, emb2):
    nb = BQ // RB
    vec = lambda: pl.BlockSpec((RB,), lambda i: (i,))
    mat = lambda: pl.BlockSpec((RB, KN), lambda i: (i, 0))
    return pl.pallas_call(
        _tc_topk_body,
        grid=(nb,),
        in_specs=[vec(), vec(), vec(),
                  pl.BlockSpec((RB, EMB), lambda i: (i, 0)),
                  pl.BlockSpec((RB, EMB), lambda i: (i, 0)),
                  pl.BlockSpec((S1, EMB), lambda i: (0, 0)),
                  pl.BlockSpec((S2, EMB), lambda i: (0, 0))],
        out_specs=[mat(), mat(), mat(), mat(), vec()],
        out_shape=[jax.ShapeDtypeStruct((BQ, KN), F32),
                   jax.ShapeDtypeStruct((BQ, KN), I32),
                   jax.ShapeDtypeStruct((BQ, KN), F32),
                   jax.ShapeDtypeStruct((BQ, KN), I32),
                   jax.ShapeDtypeStruct((BQ,), I32)],
    )(time, index1, index2, q1, q2, emb1, emb2)


def _sc_selgather_body(res_hbm, means_hbm, stds_hbm,
                       l1_hbm, l2_hbm, lm_hbm,
                       sel1_hbm, sel2_hbm, meanv_hbm, stdv_hbm,
                       lidx_v, out_v, sem):
    wid = lax.axis_index("s") * 2 + lax.axis_index("c")
    fb = wid * GPW
    qb = wid * QPW

    def do_side(l_hbm, sel_hbm):
        pltpu.sync_copy(l_hbm.at[pl.ds(fb, GPW)], lidx_v)

        def chunk(c, carry):
            pltpu.async_copy(res_hbm.at[lidx_v.at[pl.ds(c * 128, 128)]],
                             out_v.at[pl.ds(c * 128, 128)], sem).wait()
            return carry

        lax.fori_loop(0, NCH, chunk, 0)
        pltpu.sync_copy(out_v, sel_hbm.at[pl.ds(fb, GPW)])

    do_side(l1_hbm, sel1_hbm)
    do_side(l2_hbm, sel2_hbm)

    pltpu.sync_copy(lm_hbm.at[pl.ds(qb, QPW)], lidx_v.at[pl.ds(0, QPW)])
    for tab_hbm, outv_hbm in ((means_hbm, meanv_hbm), (stds_hbm, stdv_hbm)):
        pltpu.async_copy(tab_hbm.at[lidx_v.at[pl.ds(0, QPW)]],
                         out_v.at[pl.ds(0, QPW)], sem).wait()
        pltpu.sync_copy(out_v.at[pl.ds(0, QPW)], outv_hbm.at[pl.ds(qb, QPW)])


def _sc_selgather(res1d, means1d, stds1d, l1, l2, lm):
    mesh = plsc.VectorSubcoreMesh(core_axis_name="c", subcore_axis_name="s")
    fn = pl.kernel(
        _sc_selgather_body,
        mesh=mesh,
        out_type=(jax.ShapeDtypeStruct((BQ * KN,), F32),
                  jax.ShapeDtypeStruct((BQ * KN,), F32),
                  jax.ShapeDtypeStruct((BQ,), F32),
                  jax.ShapeDtypeStruct((BQ,), F32)),
        scratch_types=[pltpu.VMEM((GPW,), I32),
                       pltpu.VMEM((GPW,), F32),
                       pltpu.SemaphoreType.DMA],
    )
    return fn(res1d, means1d, stds1d, l1, l2, lm)


def _tc_mlp_body(sel1_ref, w1s_ref, sel2_ref, w2s_ref, mean_ref, std_ref,
                 w1t_ref, b1_ref, wm_ref, bm_ref, wsd_ref, bs_ref,
                 mo_ref, so_ref):
    def side_feats(sel_ref, ws_ref):
        sel = sel_ref[...]
        w = ws_ref[...]
        wsum = jnp.sum(w, axis=1, keepdims=True)
        wmean = jnp.sum(sel * w, axis=1, keepdims=True) / wsum
        mu = jnp.mean(sel, axis=1, keepdims=True)
        var = jnp.sum((sel - mu) * (sel - mu), axis=1, keepdims=True)
        sd = jnp.sqrt(var / (KN - 1))
        return wmean, wsum, sd

    f0, f1, f2 = side_feats(sel1_ref, w1s_ref)
    f3, f4, f5 = side_feats(sel2_ref, w2s_ref)
    feats = (f0, f1, f2, f3, f4, f5,
             mean_ref[...][:, None], std_ref[...][:, None])
    acc = jnp.broadcast_to(b1_ref[...][None, :], (feats[0].shape[0], HID))
    for j, f in enumerate(feats):
        acc = acc + f * w1t_ref[j, :][None, :]
    h = jnp.maximum(acc, 0.0)
    mo_ref[...] = jnp.sum(h * wm_ref[...], axis=1) + bm_ref[0]
    so_ref[...] = jnp.sum(h * wsd_ref[...], axis=1) + bs_ref[0]


def _tc_mlp(sel1, w1s, sel2, w2s, meanv, stdv, w1t, b1, wm, bm, wsd, bs):
    return pl.pallas_call(
        _tc_mlp_body,
        out_shape=[jax.ShapeDtypeStruct((BQ,), F32),
                   jax.ShapeDtypeStruct((BQ,), F32)],
    )(sel1, w1s, sel2, w2s, meanv, stdv, w1t, b1, wm, bm, wsd, bs)


def kernel(time, index1, index2, residuals, means, stds, emb1, emb2,
           W1, b1, Wm, bm, Ws, bs):
    time = time.astype(I32)
    index1 = index1.astype(I32)
    index2 = index2.astype(I32)
    q1, q2 = _sc_qgather(index1, index2, emb1, emb2)
    w1s, l1, w2s, l2, lm = _tc_topk(
        time, index1, index2, q1, q2, emb1, emb2)
    sel1f, sel2f, meanv, stdv = _sc_selgather(
        residuals.reshape(-1), means.reshape(-1), stds.reshape(-1),
        l1.reshape(-1), l2.reshape(-1), lm)
    mo, so = _tc_mlp(sel1f.reshape(BQ, KN), w1s, sel2f.reshape(BQ, KN), w2s,
                     meanv, stdv, W1.T, b1, Wm, bm, Ws, bs)
    return mo, so


# speedup vs baseline: 9.7495x; 1.1410x over previous
import functools

import jax
import jax.numpy as jnp
from jax import lax
from jax.experimental import pallas as pl
from jax.experimental.pallas import tpu as pltpu
from jax.experimental.pallas import tpu_sc as plsc

TT = 2
S1 = 2048
S2 = 2048
BQ = 4096
EMB = 256
HID = 64
KN = 20
RB = 256
NW = 32
QPW = BQ // NW
GPW = QPW * KN
NCH = GPW // 128
ROWDIV = S2 // 16
F32 = jnp.float32
I32 = jnp.int32


def _sc_qgather_body(i1_hbm, i2_hbm, emb1_hbm, emb2_hbm, q1_hbm, q2_hbm,
                     idx1_v, idx2_v, rows1_v, rows2_v, sem):
    wid = lax.axis_index("s") * 2 + lax.axis_index("c")
    base = wid * QPW
    pltpu.sync_copy(i1_hbm.at[pl.ds(base, QPW)], idx1_v)
    pltpu.sync_copy(i2_hbm.at[pl.ds(base, QPW)], idx2_v)
    cp1 = pltpu.async_copy(emb1_hbm.at[idx1_v], rows1_v, sem)
    cp2 = pltpu.async_copy(emb2_hbm.at[idx2_v], rows2_v, sem)
    cp1.wait()
    pltpu.sync_copy(rows1_v, q1_hbm.at[pl.ds(base, QPW)])
    cp2.wait()
    pltpu.sync_copy(rows2_v, q2_hbm.at[pl.ds(base, QPW)])


def _sc_qgather(index1, index2, emb1, emb2):
    mesh = plsc.VectorSubcoreMesh(core_axis_name="c", subcore_axis_name="s")
    fn = pl.kernel(
        _sc_qgather_body,
        mesh=mesh,
        out_type=(jax.ShapeDtypeStruct((BQ, EMB), F32),
                  jax.ShapeDtypeStruct((BQ, EMB), F32)),
        scratch_types=[pltpu.VMEM((QPW,), I32),
                       pltpu.VMEM((QPW,), I32),
                       pltpu.VMEM((QPW, EMB), F32),
                       pltpu.VMEM((QPW, EMB), F32),
                       pltpu.SemaphoreType.DMA],
    )
    return fn(index1, index2, emb1, emb2)


def _topk_side(q, e, self_idx):
    qn = jnp.sum(q * q, axis=1, keepdims=True)
    en = jnp.sum(e * e, axis=1)[None, :]
    g = lax.dot_general(q, e, (((1,), (1,)), ((), ())),
                        preferred_element_type=F32)
    key = 2.0 * g - (qn + en)
    cols = lax.broadcasted_iota(I32, key.shape, 1)
    key = jnp.where(cols == self_idx[:, None], -jnp.inf, key)
    ks, idxs = [], []
    for _ in range(KN):
        m = jnp.max(key, axis=1)
        msk = key == m[:, None]
        a = jnp.min(jnp.where(msk, cols, key.shape[1]), axis=1)
        ks.append(m)
        idxs.append(a)
        key = jnp.where(msk, -jnp.inf, key)
    d2sel = -jnp.stack(ks, axis=1)
    w = jnp.exp(-(jnp.sqrt(jnp.maximum(d2sel, 0.0)) + 0.001))
    return w, jnp.stack(idxs, axis=1)


def _tc_topk_body(time_ref, i1_ref, i2_ref, q1_ref, q2_ref, e1_ref, e2_ref,
                  w1_ref, l1_ref, w2_ref, l2_ref, lm_ref):
    t = time_ref[...]
    i1 = i1_ref[...]
    i2 = i2_ref[...]
    w1, n1 = _topk_side(q1_ref[...], e1_ref[...], i1)
    w2, n2 = _topk_side(q2_ref[...], e2_ref[...], i2)
    tb = (t * (S1 * S2))[:, None]
    w1_ref[...] = w1
    l1_ref[...] = tb + n1 * S2 + i2[:, None]
    w2_ref[...] = w2
    l2_ref[...] = tb + (i1 * S2)[:, None] + n2
    lm_ref[...] = t * (S1 * S2) + i1 * S2 + i2


def _tc_topk(time, index1, index2, q1, q2, emb1, emb2):
    nb = BQ // RB
    vec = lambda: pl.BlockSpec((RB,), lambda i: (i,))
    mat = lambda: pl.BlockSpec((RB, KN), lambda i: (i, 0))
    return pl.pallas_call(
        _tc_topk_body,
        grid=(nb,),
        in_specs=[vec(), vec(), vec(),
                  pl.BlockSpec((RB, EMB), lambda i: (i, 0)),
                  pl.BlockSpec((RB, EMB), lambda i: (i, 0)),
                  pl.BlockSpec((S1, EMB), lambda i: (0, 0)),
                  pl.BlockSpec((S2, EMB), lambda i: (0, 0))],
        out_specs=[mat(), mat(), mat(), mat(), vec()],
        out_shape=[jax.ShapeDtypeStruct((BQ, KN), F32),
                   jax.ShapeDtypeStruct((BQ, KN), I32),
                   jax.ShapeDtypeStruct((BQ, KN), F32),
                   jax.ShapeDtypeStruct((BQ, KN), I32),
                   jax.ShapeDtypeStruct((BQ,), I32)],
    )(time, index1, index2, q1, q2, emb1, emb2)


def _sc_selgather_body(res_hbm, means_hbm, stds_hbm,
                       l1_hbm, l2_hbm, lm_hbm,
                       sel1_hbm, sel2_hbm, meanv_hbm, stdv_hbm,
                       lidx_v, out_v, sem):
    wid = lax.axis_index("s") * 2 + lax.axis_index("c")
    fb = wid * GPW
    qb = wid * QPW

    def do_side(l_hbm, sel_hbm):
        pltpu.sync_copy(l_hbm.at[pl.ds(fb, GPW)], lidx_v)
        cps = [pltpu.async_copy(res_hbm.at[lidx_v.at[pl.ds(c * 128, 128)]],
                                out_v.at[pl.ds(c * 128, 128)], sem)
               for c in range(NCH)]
        for cp in cps:
            cp.wait()
        pltpu.sync_copy(out_v, sel_hbm.at[pl.ds(fb, GPW)])

    do_side(l1_hbm, sel1_hbm)
    do_side(l2_hbm, sel2_hbm)

    pltpu.sync_copy(lm_hbm.at[pl.ds(qb, QPW)], lidx_v.at[pl.ds(0, QPW)])
    cpm = pltpu.async_copy(means_hbm.at[lidx_v.at[pl.ds(0, QPW)]],
                           out_v.at[pl.ds(0, QPW)], sem)
    cps_ = pltpu.async_copy(stds_hbm.at[lidx_v.at[pl.ds(0, QPW)]],
                            out_v.at[pl.ds(QPW, QPW)], sem)
    cpm.wait()
    pltpu.sync_copy(out_v.at[pl.ds(0, QPW)], meanv_hbm.at[pl.ds(qb, QPW)])
    cps_.wait()
    pltpu.sync_copy(out_v.at[pl.ds(QPW, QPW)], stdv_hbm.at[pl.ds(qb, QPW)])


def _sc_selgather(res1d, means1d, stds1d, l1, l2, lm):
    mesh = plsc.VectorSubcoreMesh(core_axis_name="c", subcore_axis_name="s")
    fn = pl.kernel(
        _sc_selgather_body,
        mesh=mesh,
        out_type=(jax.ShapeDtypeStruct((BQ * KN,), F32),
                  jax.ShapeDtypeStruct((BQ * KN,), F32),
                  jax.ShapeDtypeStruct((BQ,), F32),
                  jax.ShapeDtypeStruct((BQ,), F32)),
        scratch_types=[pltpu.VMEM((GPW,), I32),
                       pltpu.VMEM((GPW,), F32),
                       pltpu.SemaphoreType.DMA],
    )
    return fn(res1d, means1d, stds1d, l1, l2, lm)


def _tc_mlp_body(sel1_ref, w1s_ref, sel2_ref, w2s_ref, mean_ref, std_ref,
                 w1t_ref, b1_ref, wm_ref, bm_ref, wsd_ref, bs_ref,
                 mo_ref, so_ref):
    def side_feats(sel_ref, ws_ref):
        sel = sel_ref[...]
        w = ws_ref[...]
        wsum = jnp.sum(w, axis=1, keepdims=True)
        wmean = jnp.sum(sel * w, axis=1, keepdims=True) / wsum
        mu = jnp.mean(sel, axis=1, keepdims=True)
        var = jnp.sum((sel - mu) * (sel - mu), axis=1, keepdims=True)
        sd = jnp.sqrt(var / (KN - 1))
        return wmean, wsum, sd

    f0, f1, f2 = side_feats(sel1_ref, w1s_ref)
    f3, f4, f5 = side_feats(sel2_ref, w2s_ref)
    feats = (f0, f1, f2, f3, f4, f5,
             mean_ref[...][:, None], std_ref[...][:, None])
    acc = jnp.broadcast_to(b1_ref[...][None, :], (feats[0].shape[0], HID))
    for j, f in enumerate(feats):
        acc = acc + f * w1t_ref[j, :][None, :]
    h = jnp.maximum(acc, 0.0)
    mo_ref[...] = jnp.sum(h * wm_ref[...], axis=1) + bm_ref[0]
    so_ref[...] = jnp.sum(h * wsd_ref[...], axis=1) + bs_ref[0]


def _tc_mlp(sel1, w1s, sel2, w2s, meanv, stdv, w1t, b1, wm, bm, wsd, bs):
    return pl.pallas_call(
        _tc_mlp_body,
        out_shape=[jax.ShapeDtypeStruct((BQ,), F32),
                   jax.ShapeDtypeStruct((BQ,), F32)],
    )(sel1, w1s, sel2, w2s, meanv, stdv, w1t, b1, wm, bm, wsd, bs)


def kernel(time, index1, index2, residuals, means, stds, emb1, emb2,
           W1, b1, Wm, bm, Ws, bs):
    time = time.astype(I32)
    index1 = index1.astype(I32)
    index2 = index2.astype(I32)
    q1, q2 = _sc_qgather(index1, index2, emb1, emb2)
    w1s, l1, w2s, l2, lm = _tc_topk(
        time, index1, index2, q1, q2, emb1, emb2)
    sel1f, sel2f, meanv, stdv = _sc_selgather(
        residuals.reshape(-1), means.reshape(-1), stds.reshape(-1),
        l1.reshape(-1), l2.reshape(-1), lm)
    mo, so = _tc_mlp(sel1f.reshape(BQ, KN), w1s, sel2f.reshape(BQ, KN), w2s,
                     meanv, stdv, W1.T, b1, Wm, bm, Ws, bs)
    return mo, so
